# Initial kernel scaffold; baseline (speedup 1.0000x reference)
import jax
import jax.numpy as jnp
from jax.experimental import pallas as pl


def kernel(kg_name_embed, eer_adj_index, eer_adj_data, r_head, r_tail, kg_name_w, kg_name_b, w_R_Left, w_R_Right, w_atten_r):
    raise NotImplementedError("write your pallas kernel here")


# speedup vs baseline: 5.1709x; 5.1709x over previous
import functools

import jax
import jax.numpy as jnp
from jax import lax
from jax.experimental import pallas as pl
from jax.experimental.pallas import tpu as pltpu
from jax.experimental.pallas import tpu_sc as plsc

KG_E = 10000
KG_R = 1000
E_DIM = 128
N_EDGES = 320000
ALPHA = 0.2
BETA1 = 0.3

NC = 2
NS = 16
NW = NC * NS
EPW = N_EDGES // NW
C = 80
NCHUNK = EPW // C
RPT = KG_E // NS


def _name_proj_body(x_ref, w_ref, b_ref, o_ref):
    o_ref[...] = (
        jnp.dot(x_ref[...], w_ref[...], preferred_element_type=jnp.float32)
        + b_ref[...]
    )


def _name_proj(x, w, b2d):
    return pl.pallas_call(
        _name_proj_body,
        grid=(10,),
        in_specs=[
            pl.BlockSpec((1000, 300), lambda i: (i, 0)),
            pl.BlockSpec((300, E_DIM), lambda i: (0, 0)),
            pl.BlockSpec((1, E_DIM), lambda i: (0, 0)),
        ],
        out_specs=pl.BlockSpec((1000, E_DIM), lambda i: (i, 0)),
        out_shape=jax.ShapeDtypeStruct((KG_E, E_DIM), jnp.float32),
    )(x, w, b2d)


def _lr_proj_body(e_ref, wl_ref, wr_ref, le_ref, re_ref):
    e = e_ref[...]
    le_ref[...] = jnp.dot(e, wl_ref[...], preferred_element_type=jnp.float32)
    re_ref[...] = jnp.dot(e, wr_ref[...], preferred_element_type=jnp.float32)


def _lr_proj(e, wl, wr):
    return pl.pallas_call(
        _lr_proj_body,
        grid=(10,),
        in_specs=[
            pl.BlockSpec((1000, E_DIM), lambda i: (i, 0)),
            pl.BlockSpec((E_DIM, E_DIM), lambda i: (0, 0)),
            pl.BlockSpec((E_DIM, E_DIM), lambda i: (0, 0)),
        ],
        out_specs=[
            pl.BlockSpec((1000, E_DIM), lambda i: (i, 0)),
            pl.BlockSpec((1000, E_DIM), lambda i: (i, 0)),
        ],
        out_shape=[
            jax.ShapeDtypeStruct((KG_E, E_DIM), jnp.float32),
            jax.ShapeDtypeStruct((KG_E, E_DIM), jnp.float32),
        ],
    )(e, wl, wr)


def _rel_tables_body(rh_ref, rt_ref, le_ref, re_ref, wal_ref, war_ref,
                     a_ref, b_ref):
    rh = rh_ref[...]
    hs = jnp.sum(rh, axis=1, keepdims=True)
    hinv = jnp.where(hs == 0.0, 0.0, 1.0 / hs)
    lr = jnp.dot(rh, le_ref[...], preferred_element_type=jnp.float32) * hinv
    a_ref[...] = jnp.maximum(lr, 0.0) * wal_ref[...]
    rt = rt_ref[...]
    ts = jnp.sum(rt, axis=1, keepdims=True)
    tinv = jnp.where(ts == 0.0, 0.0, 1.0 / ts)
    rr = jnp.dot(rt, re_ref[...], preferred_element_type=jnp.float32) * tinv
    b_ref[...] = jnp.maximum(rr, 0.0) * war_ref[...]


def _rel_tables(r_head, r_tail, le, re, wal, war):
    return pl.pallas_call(
        _rel_tables_body,
        grid=(5,),
        in_specs=[
            pl.BlockSpec((200, KG_E), lambda i: (i, 0)),
            pl.BlockSpec((200, KG_E), lambda i: (i, 0)),
            pl.BlockSpec((KG_E, E_DIM), lambda i: (0, 0)),
            pl.BlockSpec((KG_E, E_DIM), lambda i: (0, 0)),
            pl.BlockSpec((1, E_DIM), lambda i: (0, 0)),
            pl.BlockSpec((1, E_DIM), lambda i: (0, 0)),
        ],
        out_specs=[
            pl.BlockSpec((200, E_DIM), lambda i: (i, 0)),
            pl.BlockSpec((200, E_DIM), lambda i: (i, 0)),
        ],
        out_shape=[
            jax.ShapeDtypeStruct((KG_R, E_DIM), jnp.float32),
            jax.ShapeDtypeStruct((KG_R, E_DIM), jnp.float32),
        ],
    )(r_head, r_tail, le, re, wal, war)


def _pq_body(e_ref, a_ref, b_ref, p_ref, q_ref):
    e = e_ref[...]
    dn = (((1,), (1,)), ((), ()))
    p_ref[...] = lax.dot_general(e, a_ref[...], dn,
                                 preferred_element_type=jnp.float32)
    q_ref[...] = lax.dot_general(e, b_ref[...], dn,
                                 preferred_element_type=jnp.float32)


def _pq(e, a_tab, b_tab):
    return pl.pallas_call(
        _pq_body,
        grid=(10,),
        in_specs=[
            pl.BlockSpec((1000, E_DIM), lambda i: (i, 0)),
            pl.BlockSpec((KG_R, E_DIM), lambda i: (0, 0)),
            pl.BlockSpec((KG_R, E_DIM), lambda i: (0, 0)),
        ],
        out_specs=[
            pl.BlockSpec((1000, KG_R), lambda i: (i, 0)),
            pl.BlockSpec((1000, KG_R), lambda i: (i, 0)),
        ],
        out_shape=[
            jax.ShapeDtypeStruct((KG_E, KG_R), jnp.float32),
            jax.ShapeDtypeStruct((KG_E, KG_R), jnp.float32),
        ],
    )(e, a_tab, b_tab)


def _rsum_inv_body(rsum_ref, o_ref):
    rs = jnp.sum(rsum_ref[...], axis=0, keepdims=True)
    inv = jnp.where(rs == 0.0, 0.0, 1.0 / rs)
    o_ref[...] = jnp.transpose(inv)


def _rsum_inv(rowsum_partial):
    return pl.pallas_call(
        _rsum_inv_body,
        grid=(1,),
        in_specs=[pl.BlockSpec((NW, KG_E), lambda i: (0, 0))],
        out_specs=pl.BlockSpec((KG_E, 1), lambda i: (0, 0)),
        out_shape=jax.ShapeDtypeStruct((KG_E, 1), jnp.float32),
    )(rowsum_partial)


def _finalize_body(name_ref, outp_ref, rinv_ref, o_ref):
    total = outp_ref[0] + outp_ref[1]
    e_out = jnp.maximum(total * rinv_ref[...], 0.0)
    o_ref[...] = name_ref[...] + BETA1 * e_out


def _finalize(name_embed, out_partial, rowsum_partial):
    rinv = _rsum_inv(rowsum_partial)
    return pl.pallas_call(
        _finalize_body,
        grid=(10,),
        in_specs=[
            pl.BlockSpec((1000, E_DIM), lambda i: (i, 0)),
            pl.BlockSpec((NC, 1000, E_DIM), lambda i: (0, i, 0)),
            pl.BlockSpec((1000, 1), lambda i: (i, 0)),
        ],
        out_specs=pl.BlockSpec((1000, E_DIM), lambda i: (i, 0)),
        out_shape=jax.ShapeDtypeStruct((KG_E, E_DIM), jnp.float32),
    )(name_embed, out_partial, rinv)


_MESH = plsc.VectorSubcoreMesh(core_axis_name="c", subcore_axis_name="s")


@functools.partial(
    pl.kernel,
    out_type=[
        jax.ShapeDtypeStruct((NC, KG_E, E_DIM), jnp.float32),
        jax.ShapeDtypeStruct((NW, KG_E), jnp.float32),
    ],
    mesh=_MESH,
    compiler_params=pltpu.CompilerParams(use_tc_tiling_on_sc=False,
                                         needs_layout_passes=False),
    scratch_types=[
        pltpu.VMEM((C,), jnp.int32),
        pltpu.VMEM((C,), jnp.int32),
        pltpu.VMEM((C,), jnp.int32),
        pltpu.VMEM((C,), jnp.int32),
        pltpu.VMEM((C,), jnp.int32),
        pltpu.VMEM((C,), jnp.float32),
        pltpu.VMEM((C,), jnp.float32),
        pltpu.VMEM((C,), jnp.float32),
        pltpu.VMEM((C, E_DIM), jnp.float32),
        pltpu.VMEM((KG_E,), jnp.float32),
        pltpu.VMEM_SHARED((KG_E, E_DIM), jnp.float32),
        pltpu.SemaphoreType.DMA,
        pltpu.SemaphoreType.DMA,
    ],
)
def _edge_kernel(pf, qf, ent, src, dst, rel, zeros,
                 out_p, rowsum_p,
                 srcv, dstv, relv, idxp, idxq, spv, sqv, wv, rows, rsum,
                 acc, sem1, sem2):
    cid = lax.axis_index("c")
    sid = lax.axis_index("s")
    wid = cid * NS + sid

    pltpu.sync_copy(zeros.at[pl.ds(sid * RPT, RPT)],
                    acc.at[pl.ds(sid * RPT, RPT)])

    z16 = jnp.zeros((16,), jnp.float32)

    def _zero_rsum(i, carry):
        rsum[pl.ds(i * 16, 16)] = z16
        return carry

    lax.fori_loop(0, KG_E // 16, _zero_rsum, 0)
    plsc.subcore_barrier()

    base0 = wid * EPW

    def _chunk(c, carry):
        base = base0 + c * C
        pltpu.sync_copy(src.at[pl.ds(base, C)], srcv)
        pltpu.sync_copy(dst.at[pl.ds(base, C)], dstv)
        pltpu.sync_copy(rel.at[pl.ds(base, C)], relv)

        def _mkidx(i, carry2):
            s = srcv[pl.ds(i * 16, 16)]
            d = dstv[pl.ds(i * 16, 16)]
            r = relv[pl.ds(i * 16, 16)]
            idxp[pl.ds(i * 16, 16)] = s * KG_R + r
            idxq[pl.ds(i * 16, 16)] = d * KG_R + r
            return carry2

        lax.fori_loop(0, C // 16, _mkidx, 0)

        cp1 = pltpu.async_copy(pf.at[idxp], spv, sem1)
        cp2 = pltpu.async_copy(qf.at[idxq], sqv, sem2)
        cp1.wait()
        cp2.wait()

        def _mkw(i, carry2):
            s = spv[pl.ds(i * 16, 16)] + sqv[pl.ds(i * 16, 16)]
            ls = jnp.where(s > 0.0, s, ALPHA * s)
            wv[pl.ds(i * 16, 16)] = jnp.exp(-ls)
            return carry2

        lax.fori_loop(0, C // 16, _mkw, 0)

        pltpu.async_copy(ent.at[dstv], rows, sem1).wait()

        def _scale(i, carry2):
            w16 = wv[pl.ds(i * 16, 16)]
            for l in range(16):
                s = w16[l]
                e = i * 16 + l
                for k in range(E_DIM // 16):
                    rows[e, pl.ds(k * 16, 16)] = (
                        rows[e, pl.ds(k * 16, 16)] * s)
            return carry2

        lax.fori_loop(0, C // 16, _scale, 0)

        def _rsacc(i, carry2):
            plsc.addupdate_scatter(rsum, [srcv[pl.ds(i * 16, 16)]],
                                   wv[pl.ds(i * 16, 16)])
            return carry2

        lax.fori_loop(0, C // 16, _rsacc, 0)

        pltpu.sync_copy(rows, acc.at[srcv], add=True)
        return carry

    lax.fori_loop(0, NCHUNK, _chunk, 0)
    plsc.subcore_barrier()

    pltpu.sync_copy(acc.at[pl.ds(sid * RPT, RPT)],
                    out_p.at[cid, pl.ds(sid * RPT, RPT)])
    pltpu.sync_copy(rsum, rowsum_p.at[wid])


def _att_layer(ent, pflat, qflat, src, dst, rel, zeros, name_embed):
    out_p, rowsum_p = _edge_kernel(pflat, qflat, ent, src, dst, rel, zeros)
    return _finalize(name_embed, out_p, rowsum_p)


def kernel(kg_name_embed, eer_adj_index, eer_adj_data, r_head, r_tail,
           kg_name_w, kg_name_b, w_R_Left, w_R_Right, w_atten_r):
    b2d = kg_name_b.reshape(1, E_DIM)
    wal = w_atten_r[:E_DIM, 0].reshape(1, E_DIM)
    war = w_atten_r[E_DIM:, 0].reshape(1, E_DIM)
    src = eer_adj_index[0]
    dst = eer_adj_index[1]
    rel = eer_adj_data
    zeros = jnp.zeros((KG_E, E_DIM), jnp.float32)

    name_embed = _name_proj(kg_name_embed, kg_name_w, b2d)

    def layer(ent):
        le, re = _lr_proj(ent, w_R_Left, w_R_Right)
        a_tab, b_tab = _rel_tables(r_head, r_tail, le, re, wal, war)
        p, q = _pq(ent, a_tab, b_tab)
        return _att_layer(ent, p.reshape(-1), q.reshape(-1),
                          src, dst, rel, zeros, name_embed)

    gat1 = layer(name_embed)
    gat2 = layer(gat1)
    return gat2


# speedup vs baseline: 7.5890x; 1.4676x over previous
import functools

import jax
import jax.numpy as jnp
from jax import lax
from jax.experimental import pallas as pl
from jax.experimental.pallas import tpu as pltpu
from jax.experimental.pallas import tpu_sc as plsc

KG_E = 10000
KG_R = 1000
E_DIM = 128
N_EDGES = 320000
ALPHA = 0.2
BETA1 = 0.3

NC = 2
NS = 16
NW = NC * NS
EPW = N_EDGES // NW
C = 80
NCHUNK = EPW // C
RPT = KG_E // NS


def _name_proj_body(x_ref, w_ref, b_ref, o_ref):
    o_ref[...] = (
        jnp.dot(x_ref[...], w_ref[...], preferred_element_type=jnp.float32)
        + b_ref[...]
    )


def _name_proj(x, w, b2d):
    return pl.pallas_call(
        _name_proj_body,
        grid=(10,),
        in_specs=[
            pl.BlockSpec((1000, 300), lambda i: (i, 0)),
            pl.BlockSpec((300, E_DIM), lambda i: (0, 0)),
            pl.BlockSpec((1, E_DIM), lambda i: (0, 0)),
        ],
        out_specs=pl.BlockSpec((1000, E_DIM), lambda i: (i, 0)),
        out_shape=jax.ShapeDtypeStruct((KG_E, E_DIM), jnp.float32),
    )(x, w, b2d)


def _lr_proj_body(e_ref, wl_ref, wr_ref, le_ref, re_ref):
    e = e_ref[...]
    le_ref[...] = jnp.dot(e, wl_ref[...], preferred_element_type=jnp.float32)
    re_ref[...] = jnp.dot(e, wr_ref[...], preferred_element_type=jnp.float32)


def _lr_proj(e, wl, wr):
    return pl.pallas_call(
        _lr_proj_body,
        grid=(10,),
        in_specs=[
            pl.BlockSpec((1000, E_DIM), lambda i: (i, 0)),
            pl.BlockSpec((E_DIM, E_DIM), lambda i: (0, 0)),
            pl.BlockSpec((E_DIM, E_DIM), lambda i: (0, 0)),
        ],
        out_specs=[
            pl.BlockSpec((1000, E_DIM), lambda i: (i, 0)),
            pl.BlockSpec((1000, E_DIM), lambda i: (i, 0)),
        ],
        out_shape=[
            jax.ShapeDtypeStruct((KG_E, E_DIM), jnp.float32),
            jax.ShapeDtypeStruct((KG_E, E_DIM), jnp.float32),
        ],
    )(e, wl, wr)


def _rel_tables_body(rh_ref, rt_ref, le_ref, re_ref, wal_ref, war_ref,
                     a_ref, b_ref):
    rh = rh_ref[...]
    hs = jnp.sum(rh, axis=1, keepdims=True)
    hinv = jnp.where(hs == 0.0, 0.0, 1.0 / hs)
    lr = jnp.dot(rh, le_ref[...], preferred_element_type=jnp.float32) * hinv
    a_ref[...] = jnp.maximum(lr, 0.0) * wal_ref[...]
    rt = rt_ref[...]
    ts = jnp.sum(rt, axis=1, keepdims=True)
    tinv = jnp.where(ts == 0.0, 0.0, 1.0 / ts)
    rr = jnp.dot(rt, re_ref[...], preferred_element_type=jnp.float32) * tinv
    b_ref[...] = jnp.maximum(rr, 0.0) * war_ref[...]


def _rel_tables(r_head, r_tail, le, re, wal, war):
    return pl.pallas_call(
        _rel_tables_body,
        grid=(5,),
        in_specs=[
            pl.BlockSpec((200, KG_E), lambda i: (i, 0)),
            pl.BlockSpec((200, KG_E), lambda i: (i, 0)),
            pl.BlockSpec((KG_E, E_DIM), lambda i: (0, 0)),
            pl.BlockSpec((KG_E, E_DIM), lambda i: (0, 0)),
            pl.BlockSpec((1, E_DIM), lambda i: (0, 0)),
            pl.BlockSpec((1, E_DIM), lambda i: (0, 0)),
        ],
        out_specs=[
            pl.BlockSpec((200, E_DIM), lambda i: (i, 0)),
            pl.BlockSpec((200, E_DIM), lambda i: (i, 0)),
        ],
        out_shape=[
            jax.ShapeDtypeStruct((KG_R, E_DIM), jnp.float32),
            jax.ShapeDtypeStruct((KG_R, E_DIM), jnp.float32),
        ],
    )(r_head, r_tail, le, re, wal, war)


def _pq_body(e_ref, a_ref, b_ref, p_ref, q_ref):
    e = e_ref[...]
    dn = (((1,), (1,)), ((), ()))
    p_ref[...] = lax.dot_general(e, a_ref[...], dn,
                                 preferred_element_type=jnp.float32)
    q_ref[...] = lax.dot_general(e, b_ref[...], dn,
                                 preferred_element_type=jnp.float32)


def _pq(e, a_tab, b_tab):
    return pl.pallas_call(
        _pq_body,
        grid=(10,),
        in_specs=[
            pl.BlockSpec((1000, E_DIM), lambda i: (i, 0)),
            pl.BlockSpec((KG_R, E_DIM), lambda i: (0, 0)),
            pl.BlockSpec((KG_R, E_DIM), lambda i: (0, 0)),
        ],
        out_specs=[
            pl.BlockSpec((1000, KG_R), lambda i: (i, 0)),
            pl.BlockSpec((1000, KG_R), lambda i: (i, 0)),
        ],
        out_shape=[
            jax.ShapeDtypeStruct((KG_E, KG_R), jnp.float32),
            jax.ShapeDtypeStruct((KG_E, KG_R), jnp.float32),
        ],
    )(e, a_tab, b_tab)


def _rsum_inv_body(rsum_ref, o_ref):
    rs = jnp.sum(rsum_ref[...], axis=0, keepdims=True)
    inv = jnp.where(rs == 0.0, 0.0, 1.0 / rs)
    o_ref[...] = jnp.transpose(inv)


def _rsum_inv(rowsum_partial):
    return pl.pallas_call(
        _rsum_inv_body,
        grid=(1,),
        in_specs=[pl.BlockSpec((NW, KG_E), lambda i: (0, 0))],
        out_specs=pl.BlockSpec((KG_E, 1), lambda i: (0, 0)),
        out_shape=jax.ShapeDtypeStruct((KG_E, 1), jnp.float32),
    )(rowsum_partial)


def _finalize_body(name_ref, outp_ref, rinv_ref, o_ref):
    total = outp_ref[0] + outp_ref[1]
    e_out = jnp.maximum(total * rinv_ref[...], 0.0)
    o_ref[...] = name_ref[...] + BETA1 * e_out


def _finalize(name_embed, out_partial, rowsum_partial):
    rinv = _rsum_inv(rowsum_partial)
    return pl.pallas_call(
        _finalize_body,
        grid=(10,),
        in_specs=[
            pl.BlockSpec((1000, E_DIM), lambda i: (i, 0)),
            pl.BlockSpec((NC, 1000, E_DIM), lambda i: (0, i, 0)),
            pl.BlockSpec((1000, 1), lambda i: (i, 0)),
        ],
        out_specs=pl.BlockSpec((1000, E_DIM), lambda i: (i, 0)),
        out_shape=jax.ShapeDtypeStruct((KG_E, E_DIM), jnp.float32),
    )(name_embed, out_partial, rinv)


_MESH = plsc.VectorSubcoreMesh(core_axis_name="c", subcore_axis_name="s")


@functools.partial(
    pl.kernel,
    out_type=[
        jax.ShapeDtypeStruct((NC, KG_E, E_DIM), jnp.float32),
        jax.ShapeDtypeStruct((NW, KG_E), jnp.float32),
    ],
    mesh=_MESH,
    compiler_params=pltpu.CompilerParams(use_tc_tiling_on_sc=False,
                                         needs_layout_passes=False),
    scratch_types=[
        pltpu.VMEM((C,), jnp.int32),
        pltpu.VMEM((C,), jnp.int32),
        pltpu.VMEM((C,), jnp.int32),
        pltpu.VMEM((C,), jnp.int32),
        pltpu.VMEM((C,), jnp.int32),
        pltpu.VMEM((C,), jnp.float32),
        pltpu.VMEM((C,), jnp.float32),
        pltpu.VMEM((C, E_DIM), jnp.float32),
        pltpu.VMEM((C,), jnp.int32),
        pltpu.VMEM((C,), jnp.int32),
        pltpu.VMEM((C,), jnp.int32),
        pltpu.VMEM((C,), jnp.int32),
        pltpu.VMEM((C,), jnp.int32),
        pltpu.VMEM((C,), jnp.float32),
        pltpu.VMEM((C,), jnp.float32),
        pltpu.VMEM((C, E_DIM), jnp.float32),
        pltpu.VMEM((C,), jnp.float32),
        pltpu.VMEM((KG_E,), jnp.float32),
        pltpu.VMEM_SHARED((KG_E, E_DIM), jnp.float32),
        pltpu.SemaphoreType.DMA,
        pltpu.SemaphoreType.DMA,
        pltpu.SemaphoreType.DMA,
        pltpu.SemaphoreType.DMA,
        pltpu.SemaphoreType.DMA,
        pltpu.SemaphoreType.DMA,
        pltpu.SemaphoreType.DMA,
        pltpu.SemaphoreType.DMA,
    ],
)
def _edge_kernel(pf, qf, ent, src, dst, rel, zeros,
                 out_p, rowsum_p,
                 srcA, dstA, relA, idxpA, idxqA, spA, sqA, rowsA,
                 srcB, dstB, relB, idxpB, idxqB, spB, sqB, rowsB,
                 wv, rsum, acc,
                 semlinA, semspA, semsqA, semrowsA,
                 semlinB, semspB, semsqB, semrowsB):
    cid = lax.axis_index("c")
    sid = lax.axis_index("s")
    wid = cid * NS + sid

    A = dict(src=srcA, dst=dstA, rel=relA, idxp=idxpA, idxq=idxqA,
             sp=spA, sq=sqA, rows=rowsA, semlin=semlinA, semsp=semspA,
             semsq=semsqA, semrows=semrowsA)
    B = dict(src=srcB, dst=dstB, rel=relB, idxp=idxpB, idxq=idxqB,
             sp=spB, sq=sqB, rows=rowsB, semlin=semlinB, semsp=semspB,
             semsq=semsqB, semrows=semrowsB)

    pltpu.sync_copy(zeros.at[pl.ds(sid * RPT, RPT)],
                    acc.at[pl.ds(sid * RPT, RPT)])

    z16 = jnp.zeros((16,), jnp.float32)

    def _zero_rsum(i, carry):
        rsum[pl.ds(i * 16, 16)] = z16
        return carry

    lax.fori_loop(0, KG_E // 16, _zero_rsum, 0)
    plsc.subcore_barrier()

    base0 = wid * EPW

    def _mkidx(buf):
        def body(i, carry):
            s = buf['src'][pl.ds(i * 16, 16)]
            d = buf['dst'][pl.ds(i * 16, 16)]
            r = buf['rel'][pl.ds(i * 16, 16)]
            buf['idxp'][pl.ds(i * 16, 16)] = s * KG_R + r
            buf['idxq'][pl.ds(i * 16, 16)] = d * KG_R + r
            return carry
        lax.fori_loop(0, C // 16, body, 0)

    def _start_lin(c, buf):
        base = base0 + c * C
        pltpu.async_copy(src.at[pl.ds(base, C)], buf['src'], buf['semlin'])
        pltpu.async_copy(dst.at[pl.ds(base, C)], buf['dst'], buf['semlin'])
        pltpu.async_copy(rel.at[pl.ds(base, C)], buf['rel'], buf['semlin'])

    def _wait_lin(buf):
        pltpu.make_async_copy(src.at[pl.ds(0, C)], buf['src'],
                              buf['semlin']).wait()
        pltpu.make_async_copy(dst.at[pl.ds(0, C)], buf['dst'],
                              buf['semlin']).wait()
        pltpu.make_async_copy(rel.at[pl.ds(0, C)], buf['rel'],
                              buf['semlin']).wait()

    def _start_gathers(buf):
        pltpu.async_copy(pf.at[buf['idxp']], buf['sp'], buf['semsp'])
        pltpu.async_copy(qf.at[buf['idxq']], buf['sq'], buf['semsq'])
        pltpu.async_copy(ent.at[buf['dst']], buf['rows'], buf['semrows'])

    def _wait_gathers(buf):
        pltpu.make_async_copy(pf.at[buf['idxp']], buf['sp'],
                              buf['semsp']).wait()
        pltpu.make_async_copy(qf.at[buf['idxq']], buf['sq'],
                              buf['semsq']).wait()
        pltpu.make_async_copy(ent.at[buf['dst']], buf['rows'],
                              buf['semrows']).wait()

    def _do_chunk(c, cur, nxt):
        _wait_gathers(cur)

        def _mkw(i, carry2):
            s = cur['sp'][pl.ds(i * 16, 16)] + cur['sq'][pl.ds(i * 16, 16)]
            ls = jnp.where(s > 0.0, s, ALPHA * s)
            wv[pl.ds(i * 16, 16)] = jnp.exp(-ls)
            return carry2

        lax.fori_loop(0, C // 16, _mkw, 0)

        def _scale(i, carry2):
            w16 = wv[pl.ds(i * 16, 16)]
            for l in range(16):
                s = w16[l]
                e = i * 16 + l
                for k in range(E_DIM // 16):
                    cur['rows'][e, pl.ds(k * 16, 16)] = (
                        cur['rows'][e, pl.ds(k * 16, 16)] * s)
            return carry2

        lax.fori_loop(0, C // 16, _scale, 0)

        def _rsacc(i, carry2):
            plsc.addupdate_scatter(rsum, [cur['src'][pl.ds(i * 16, 16)]],
                                   wv[pl.ds(i * 16, 16)])
            return carry2

        lax.fori_loop(0, C // 16, _rsacc, 0)

        pltpu.sync_copy(cur['rows'], acc.at[cur['src']], add=True)

        @pl.when(c + 1 < NCHUNK)
        def _():
            _wait_lin(nxt)
            _mkidx(nxt)
            _start_gathers(nxt)

        @pl.when(c + 2 < NCHUNK)
        def _():
            _start_lin(c + 2, cur)

    _start_lin(0, A)
    _wait_lin(A)
    _mkidx(A)
    _start_gathers(A)
    _start_lin(1, B)

    def _pair(j, carry):
        _do_chunk(2 * j, A, B)
        _do_chunk(2 * j + 1, B, A)
        return carry

    lax.fori_loop(0, NCHUNK // 2, _pair, 0)
    if NCHUNK % 2 == 1:
        _do_chunk(NCHUNK - 1, A, B)

    plsc.subcore_barrier()

    pltpu.sync_copy(acc.at[pl.ds(sid * RPT, RPT)],
                    out_p.at[cid, pl.ds(sid * RPT, RPT)])
    pltpu.sync_copy(rsum, rowsum_p.at[wid])


def _att_layer(ent, pflat, qflat, src, dst, rel, zeros, name_embed):
    out_p, rowsum_p = _edge_kernel(pflat, qflat, ent, src, dst, rel, zeros)
    return _finalize(name_embed, out_p, rowsum_p)


def kernel(kg_name_embed, eer_adj_index, eer_adj_data, r_head, r_tail,
           kg_name_w, kg_name_b, w_R_Left, w_R_Right, w_atten_r):
    b2d = kg_name_b.reshape(1, E_DIM)
    wal = w_atten_r[:E_DIM, 0].reshape(1, E_DIM)
    war = w_atten_r[E_DIM:, 0].reshape(1, E_DIM)
    src = eer_adj_index[0]
    dst = eer_adj_index[1]
    rel = eer_adj_data
    zeros = jnp.zeros((KG_E, E_DIM), jnp.float32)

    name_embed = _name_proj(kg_name_embed, kg_name_w, b2d)

    def layer(ent):
        le, re = _lr_proj(ent, w_R_Left, w_R_Right)
        a_tab, b_tab = _rel_tables(r_head, r_tail, le, re, wal, war)
        p, q = _pq(ent, a_tab, b_tab)
        return _att_layer(ent, p.reshape(-1), q.reshape(-1),
                          src, dst, rel, zeros, name_embed)

    gat1 = layer(name_embed)
    gat2 = layer(gat1)
    return gat2


# speedup vs baseline: 8.4710x; 1.1162x over previous
import functools

import jax
import jax.numpy as jnp
from jax import lax
from jax.experimental import pallas as pl
from jax.experimental.pallas import tpu as pltpu
from jax.experimental.pallas import tpu_sc as plsc

KG_E = 10000
KG_R = 1000
E_DIM = 128
N_EDGES = 320000
ALPHA = 0.2
BETA1 = 0.3

NC = 2
NS = 16
NW = NC * NS
EPW = N_EDGES // NW
C = 80
NCHUNK = EPW // C
RPT = KG_E // NS


def _name_proj_body(x_ref, w_ref, b_ref, o_ref):
    o_ref[...] = (
        jnp.dot(x_ref[...], w_ref[...], preferred_element_type=jnp.float32)
        + b_ref[...]
    )


def _name_proj(x, w, b2d):
    return pl.pallas_call(
        _name_proj_body,
        grid=(10,),
        in_specs=[
            pl.BlockSpec((1000, 300), lambda i: (i, 0)),
            pl.BlockSpec((300, E_DIM), lambda i: (0, 0)),
            pl.BlockSpec((1, E_DIM), lambda i: (0, 0)),
        ],
        out_specs=pl.BlockSpec((1000, E_DIM), lambda i: (i, 0)),
        out_shape=jax.ShapeDtypeStruct((KG_E, E_DIM), jnp.float32),
    )(x, w, b2d)


def _lr_proj_body(e_ref, wl_ref, wr_ref, le_ref, re_ref):
    e = e_ref[...]
    le_ref[...] = jnp.dot(e, wl_ref[...], preferred_element_type=jnp.float32)
    re_ref[...] = jnp.dot(e, wr_ref[...], preferred_element_type=jnp.float32)


def _lr_proj(e, wl, wr):
    return pl.pallas_call(
        _lr_proj_body,
        grid=(10,),
        in_specs=[
            pl.BlockSpec((1000, E_DIM), lambda i: (i, 0)),
            pl.BlockSpec((E_DIM, E_DIM), lambda i: (0, 0)),
            pl.BlockSpec((E_DIM, E_DIM), lambda i: (0, 0)),
        ],
        out_specs=[
            pl.BlockSpec((1000, E_DIM), lambda i: (i, 0)),
            pl.BlockSpec((1000, E_DIM), lambda i: (i, 0)),
        ],
        out_shape=[
            jax.ShapeDtypeStruct((KG_E, E_DIM), jnp.float32),
            jax.ShapeDtypeStruct((KG_E, E_DIM), jnp.float32),
        ],
    )(e, wl, wr)


def _rel_tables_body(rh_ref, rt_ref, le_ref, re_ref, wal_ref, war_ref,
                     a_ref, b_ref):
    rh = rh_ref[...]
    hs = jnp.sum(rh, axis=1, keepdims=True)
    hinv = jnp.where(hs == 0.0, 0.0, 1.0 / hs)
    lr = jnp.dot(rh, le_ref[...], preferred_element_type=jnp.float32) * hinv
    a_ref[...] = jnp.maximum(lr, 0.0) * wal_ref[...]
    rt = rt_ref[...]
    ts = jnp.sum(rt, axis=1, keepdims=True)
    tinv = jnp.where(ts == 0.0, 0.0, 1.0 / ts)
    rr = jnp.dot(rt, re_ref[...], preferred_element_type=jnp.float32) * tinv
    b_ref[...] = jnp.maximum(rr, 0.0) * war_ref[...]


def _rel_tables(r_head, r_tail, le, re, wal, war):
    return pl.pallas_call(
        _rel_tables_body,
        grid=(5,),
        in_specs=[
            pl.BlockSpec((200, KG_E), lambda i: (i, 0)),
            pl.BlockSpec((200, KG_E), lambda i: (i, 0)),
            pl.BlockSpec((KG_E, E_DIM), lambda i: (0, 0)),
            pl.BlockSpec((KG_E, E_DIM), lambda i: (0, 0)),
            pl.BlockSpec((1, E_DIM), lambda i: (0, 0)),
            pl.BlockSpec((1, E_DIM), lambda i: (0, 0)),
        ],
        out_specs=[
            pl.BlockSpec((200, E_DIM), lambda i: (i, 0)),
            pl.BlockSpec((200, E_DIM), lambda i: (i, 0)),
        ],
        out_shape=[
            jax.ShapeDtypeStruct((KG_R, E_DIM), jnp.float32),
            jax.ShapeDtypeStruct((KG_R, E_DIM), jnp.float32),
        ],
    )(r_head, r_tail, le, re, wal, war)


def _pq_body(e_ref, a_ref, b_ref, p_ref, q_ref):
    e = e_ref[...]
    dn = (((1,), (1,)), ((), ()))
    p_ref[...] = lax.dot_general(e, a_ref[...], dn,
                                 preferred_element_type=jnp.float32)
    q_ref[...] = lax.dot_general(e, b_ref[...], dn,
                                 preferred_element_type=jnp.float32)


def _pq(e, a_tab, b_tab):
    return pl.pallas_call(
        _pq_body,
        grid=(10,),
        in_specs=[
            pl.BlockSpec((1000, E_DIM), lambda i: (i, 0)),
            pl.BlockSpec((KG_R, E_DIM), lambda i: (0, 0)),
            pl.BlockSpec((KG_R, E_DIM), lambda i: (0, 0)),
        ],
        out_specs=[
            pl.BlockSpec((1000, KG_R), lambda i: (i, 0)),
            pl.BlockSpec((1000, KG_R), lambda i: (i, 0)),
        ],
        out_shape=[
            jax.ShapeDtypeStruct((KG_E, KG_R), jnp.float32),
            jax.ShapeDtypeStruct((KG_E, KG_R), jnp.float32),
        ],
    )(e, a_tab, b_tab)


def _rsum_inv_body(rsum_ref, o_ref):
    rs = jnp.sum(rsum_ref[...], axis=0, keepdims=True)
    inv = jnp.where(rs == 0.0, 0.0, 1.0 / rs)
    o_ref[...] = jnp.transpose(inv)


def _rsum_inv(rowsum_partial):
    return pl.pallas_call(
        _rsum_inv_body,
        grid=(1,),
        in_specs=[pl.BlockSpec((NW, KG_E), lambda i: (0, 0))],
        out_specs=pl.BlockSpec((KG_E, 1), lambda i: (0, 0)),
        out_shape=jax.ShapeDtypeStruct((KG_E, 1), jnp.float32),
    )(rowsum_partial)


def _finalize_body(name_ref, outp_ref, rinv_ref, o_ref):
    total = outp_ref[0] + outp_ref[1]
    e_out = jnp.maximum(total * rinv_ref[...], 0.0)
    o_ref[...] = name_ref[...] + BETA1 * e_out


def _finalize(name_embed, out_partial, rowsum_partial):
    rinv = _rsum_inv(rowsum_partial)
    return pl.pallas_call(
        _finalize_body,
        grid=(10,),
        in_specs=[
            pl.BlockSpec((1000, E_DIM), lambda i: (i, 0)),
            pl.BlockSpec((NC, 1000, E_DIM), lambda i: (0, i, 0)),
            pl.BlockSpec((1000, 1), lambda i: (i, 0)),
        ],
        out_specs=pl.BlockSpec((1000, E_DIM), lambda i: (i, 0)),
        out_shape=jax.ShapeDtypeStruct((KG_E, E_DIM), jnp.float32),
    )(name_embed, out_partial, rinv)


_MESH = plsc.VectorSubcoreMesh(core_axis_name="c", subcore_axis_name="s")


@functools.partial(
    pl.kernel,
    out_type=[
        jax.ShapeDtypeStruct((NC, KG_E, E_DIM), jnp.float32),
        jax.ShapeDtypeStruct((NW, KG_E), jnp.float32),
    ],
    mesh=_MESH,
    compiler_params=pltpu.CompilerParams(use_tc_tiling_on_sc=False,
                                         needs_layout_passes=False),
    scratch_types=[
        pltpu.VMEM((C,), jnp.int32),
        pltpu.VMEM((C,), jnp.int32),
        pltpu.VMEM((C,), jnp.int32),
        pltpu.VMEM((C,), jnp.int32),
        pltpu.VMEM((C,), jnp.int32),
        pltpu.VMEM((C,), jnp.float32),
        pltpu.VMEM((C,), jnp.float32),
        pltpu.VMEM((C, E_DIM), jnp.float32),
        pltpu.VMEM((C,), jnp.int32),
        pltpu.VMEM((C,), jnp.int32),
        pltpu.VMEM((C,), jnp.int32),
        pltpu.VMEM((C,), jnp.int32),
        pltpu.VMEM((C,), jnp.int32),
        pltpu.VMEM((C,), jnp.float32),
        pltpu.VMEM((C,), jnp.float32),
        pltpu.VMEM((C, E_DIM), jnp.float32),
        pltpu.VMEM((C,), jnp.float32),
        pltpu.VMEM((C,), jnp.int32),
        pltpu.VMEM((C,), jnp.int32),
        pltpu.VMEM((KG_E,), jnp.float32),
        pltpu.VMEM_SHARED((KG_E, E_DIM), jnp.float32),
        pltpu.SemaphoreType.DMA,
        pltpu.SemaphoreType.DMA,
        pltpu.SemaphoreType.DMA,
        pltpu.SemaphoreType.DMA,
        pltpu.SemaphoreType.DMA,
        pltpu.SemaphoreType.DMA,
        pltpu.SemaphoreType.DMA,
        pltpu.SemaphoreType.DMA,
        pltpu.SemaphoreType.DMA,
        pltpu.SemaphoreType.DMA,
    ],
)
def _edge_kernel(pf, qf, ent, src, dst, rel, zeros,
                 out_p, rowsum_p,
                 srcA, dstA, relA, idxpA, idxqA, spA, sqA, rowsA,
                 srcB, dstB, relB, idxpB, idxqB, spB, sqB, rowsB,
                 wv, sidxA, sidxB, rsum, acc,
                 semlinA, semspA, semsqA, semrowsA, semaccA,
                 semlinB, semspB, semsqB, semrowsB, semaccB):
    cid = lax.axis_index("c")
    sid = lax.axis_index("s")
    wid = cid * NS + sid

    A = dict(src=srcA, dst=dstA, rel=relA, idxp=idxpA, idxq=idxqA,
             sp=spA, sq=sqA, rows=rowsA, sidx=sidxA, semlin=semlinA,
             semsp=semspA, semsq=semsqA, semrows=semrowsA, semacc=semaccA)
    B = dict(src=srcB, dst=dstB, rel=relB, idxp=idxpB, idxq=idxqB,
             sp=spB, sq=sqB, rows=rowsB, sidx=sidxB, semlin=semlinB,
             semsp=semspB, semsq=semsqB, semrows=semrowsB, semacc=semaccB)

    pltpu.sync_copy(zeros.at[pl.ds(sid * RPT, RPT)],
                    acc.at[pl.ds(sid * RPT, RPT)])

    z16 = jnp.zeros((16,), jnp.float32)

    def _zero_rsum(i, carry):
        rsum[pl.ds(i * 16, 16)] = z16
        return carry

    lax.fori_loop(0, KG_E // 16, _zero_rsum, 0)
    plsc.subcore_barrier()

    base0 = wid * EPW

    def _mkidx(buf):
        def body(i, carry):
            s = buf['src'][pl.ds(i * 16, 16)]
            d = buf['dst'][pl.ds(i * 16, 16)]
            r = buf['rel'][pl.ds(i * 16, 16)]
            buf['idxp'][pl.ds(i * 16, 16)] = s * KG_R + r
            buf['idxq'][pl.ds(i * 16, 16)] = d * KG_R + r
            return carry
        lax.fori_loop(0, C // 16, body, 0)

    def _start_lin(c, buf):
        base = base0 + c * C
        pltpu.async_copy(src.at[pl.ds(base, C)], buf['src'], buf['semlin'])
        pltpu.async_copy(dst.at[pl.ds(base, C)], buf['dst'], buf['semlin'])
        pltpu.async_copy(rel.at[pl.ds(base, C)], buf['rel'], buf['semlin'])

    def _wait_lin(buf):
        pltpu.make_async_copy(src.at[pl.ds(0, C)], buf['src'],
                              buf['semlin']).wait()
        pltpu.make_async_copy(dst.at[pl.ds(0, C)], buf['dst'],
                              buf['semlin']).wait()
        pltpu.make_async_copy(rel.at[pl.ds(0, C)], buf['rel'],
                              buf['semlin']).wait()

    def _start_gathers(buf):
        pltpu.async_copy(pf.at[buf['idxp']], buf['sp'], buf['semsp'])
        pltpu.async_copy(qf.at[buf['idxq']], buf['sq'], buf['semsq'])
        pltpu.async_copy(ent.at[buf['dst']], buf['rows'], buf['semrows'])

    def _wait_gathers(buf):
        pltpu.make_async_copy(pf.at[buf['idxp']], buf['sp'],
                              buf['semsp']).wait()
        pltpu.make_async_copy(qf.at[buf['idxq']], buf['sq'],
                              buf['semsq']).wait()
        pltpu.make_async_copy(ent.at[buf['dst']], buf['rows'],
                              buf['semrows']).wait()

    def _do_chunk(c, cur, nxt):
        _wait_gathers(cur)

        def _mkw(i, carry2):
            s = cur['sp'][pl.ds(i * 16, 16)] + cur['sq'][pl.ds(i * 16, 16)]
            ls = jnp.where(s > 0.0, s, ALPHA * s)
            wv[pl.ds(i * 16, 16)] = jnp.exp(-ls)
            return carry2

        lax.fori_loop(0, C // 16, _mkw, 0)

        def _scale(i, carry2):
            w16 = wv[pl.ds(i * 16, 16)]
            for l in range(16):
                s = w16[l]
                e = i * 16 + l
                for k in range(E_DIM // 16):
                    cur['rows'][e, pl.ds(k * 16, 16)] = (
                        cur['rows'][e, pl.ds(k * 16, 16)] * s)
            return carry2

        lax.fori_loop(0, C // 16, _scale, 0)

        def _rsacc(i, carry2):
            s16 = cur['src'][pl.ds(i * 16, 16)]
            plsc.addupdate_scatter(rsum, [s16], wv[pl.ds(i * 16, 16)])
            cur['sidx'][pl.ds(i * 16, 16)] = s16
            return carry2

        lax.fori_loop(0, C // 16, _rsacc, 0)

        @pl.when(c >= 1)
        def _():
            pltpu.make_async_copy(nxt['rows'], acc.at[nxt['sidx']],
                                  nxt['semacc']).wait()

        pltpu.async_copy(cur['rows'], acc.at[cur['sidx']], cur['semacc'],
                         add=True)

        @pl.when(c + 1 < NCHUNK)
        def _():
            _wait_lin(nxt)
            _mkidx(nxt)
            _start_gathers(nxt)

        @pl.when(c + 2 < NCHUNK)
        def _():
            _start_lin(c + 2, cur)

    _start_lin(0, A)
    _wait_lin(A)
    _mkidx(A)
    _start_gathers(A)
    _start_lin(1, B)

    def _pair(j, carry):
        _do_chunk(2 * j, A, B)
        _do_chunk(2 * j + 1, B, A)
        return carry

    lax.fori_loop(0, NCHUNK // 2, _pair, 0)
    if NCHUNK % 2 == 1:
        _do_chunk(NCHUNK - 1, A, B)

    pltpu.make_async_copy(A['rows'], acc.at[A['sidx']], A['semacc']).wait()

    plsc.subcore_barrier()

    pltpu.sync_copy(acc.at[pl.ds(sid * RPT, RPT)],
                    out_p.at[cid, pl.ds(sid * RPT, RPT)])
    pltpu.sync_copy(rsum, rowsum_p.at[wid])


def _att_layer(ent, pflat, qflat, src, dst, rel, zeros, name_embed):
    out_p, rowsum_p = _edge_kernel(pflat, qflat, ent, src, dst, rel, zeros)
    return _finalize(name_embed, out_p, rowsum_p)


def kernel(kg_name_embed, eer_adj_index, eer_adj_data, r_head, r_tail,
           kg_name_w, kg_name_b, w_R_Left, w_R_Right, w_atten_r):
    b2d = kg_name_b.reshape(1, E_DIM)
    wal = w_atten_r[:E_DIM, 0].reshape(1, E_DIM)
    war = w_atten_r[E_DIM:, 0].reshape(1, E_DIM)
    src = eer_adj_index[0]
    dst = eer_adj_index[1]
    rel = eer_adj_data
    zeros = jnp.zeros((KG_E, E_DIM), jnp.float32)

    name_embed = _name_proj(kg_name_embed, kg_name_w, b2d)

    def layer(ent):
        le, re = _lr_proj(ent, w_R_Left, w_R_Right)
        a_tab, b_tab = _rel_tables(r_head, r_tail, le, re, wal, war)
        p, q = _pq(ent, a_tab, b_tab)
        return _att_layer(ent, p.reshape(-1), q.reshape(-1),
                          src, dst, rel, zeros, name_embed)

    gat1 = layer(name_embed)
    gat2 = layer(gat1)
    return gat2


# speedup vs baseline: 10.6543x; 1.2577x over previous
import functools

import jax
import jax.numpy as jnp
from jax import lax
from jax.experimental import pallas as pl
from jax.experimental.pallas import tpu as pltpu
from jax.experimental.pallas import tpu_sc as plsc

KG_E = 10000
KG_R = 1000
E_DIM = 128
N_EDGES = 320000
ALPHA = 0.2
BETA1 = 0.3

NC = 2
NS = 16
NW = NC * NS
EPW = N_EDGES // NW
C = 80
NCHUNK = EPW // C
RPT = KG_E // NS


def _name_proj_body(x_ref, w_ref, b_ref, o_ref):
    o_ref[...] = (
        jnp.dot(x_ref[...], w_ref[...], preferred_element_type=jnp.float32)
        + b_ref[...]
    )


def _name_proj(x, w, b2d):
    return pl.pallas_call(
        _name_proj_body,
        grid=(10,),
        in_specs=[
            pl.BlockSpec((1000, 300), lambda i: (i, 0)),
            pl.BlockSpec((300, E_DIM), lambda i: (0, 0)),
            pl.BlockSpec((1, E_DIM), lambda i: (0, 0)),
        ],
        out_specs=pl.BlockSpec((1000, E_DIM), lambda i: (i, 0)),
        out_shape=jax.ShapeDtypeStruct((KG_E, E_DIM), jnp.float32),
    )(x, w, b2d)


def _lr_proj_body(e_ref, wl_ref, wr_ref, le_ref, re_ref):
    e = e_ref[...]
    le_ref[...] = jnp.dot(e, wl_ref[...], preferred_element_type=jnp.float32)
    re_ref[...] = jnp.dot(e, wr_ref[...], preferred_element_type=jnp.float32)


def _lr_proj(e, wl, wr):
    return pl.pallas_call(
        _lr_proj_body,
        grid=(10,),
        in_specs=[
            pl.BlockSpec((1000, E_DIM), lambda i: (i, 0)),
            pl.BlockSpec((E_DIM, E_DIM), lambda i: (0, 0)),
            pl.BlockSpec((E_DIM, E_DIM), lambda i: (0, 0)),
        ],
        out_specs=[
            pl.BlockSpec((1000, E_DIM), lambda i: (i, 0)),
            pl.BlockSpec((1000, E_DIM), lambda i: (i, 0)),
        ],
        out_shape=[
            jax.ShapeDtypeStruct((KG_E, E_DIM), jnp.float32),
            jax.ShapeDtypeStruct((KG_E, E_DIM), jnp.float32),
        ],
    )(e, wl, wr)


def _rel_tables_body(rh_ref, rt_ref, le_ref, re_ref, wal_ref, war_ref,
                     a_ref, b_ref):
    rh = rh_ref[...]
    hs = jnp.sum(rh, axis=1, keepdims=True)
    hinv = jnp.where(hs == 0.0, 0.0, 1.0 / hs)
    lr = jnp.dot(rh, le_ref[...], preferred_element_type=jnp.float32) * hinv
    a_ref[...] = jnp.maximum(lr, 0.0) * wal_ref[...]
    rt = rt_ref[...]
    ts = jnp.sum(rt, axis=1, keepdims=True)
    tinv = jnp.where(ts == 0.0, 0.0, 1.0 / ts)
    rr = jnp.dot(rt, re_ref[...], preferred_element_type=jnp.float32) * tinv
    b_ref[...] = jnp.maximum(rr, 0.0) * war_ref[...]


def _rel_tables(r_head, r_tail, le, re, wal, war):
    return pl.pallas_call(
        _rel_tables_body,
        grid=(5,),
        in_specs=[
            pl.BlockSpec((200, KG_E), lambda i: (i, 0)),
            pl.BlockSpec((200, KG_E), lambda i: (i, 0)),
            pl.BlockSpec((KG_E, E_DIM), lambda i: (0, 0)),
            pl.BlockSpec((KG_E, E_DIM), lambda i: (0, 0)),
            pl.BlockSpec((1, E_DIM), lambda i: (0, 0)),
            pl.BlockSpec((1, E_DIM), lambda i: (0, 0)),
        ],
        out_specs=[
            pl.BlockSpec((200, E_DIM), lambda i: (i, 0)),
            pl.BlockSpec((200, E_DIM), lambda i: (i, 0)),
        ],
        out_shape=[
            jax.ShapeDtypeStruct((KG_R, E_DIM), jnp.float32),
            jax.ShapeDtypeStruct((KG_R, E_DIM), jnp.float32),
        ],
    )(r_head, r_tail, le, re, wal, war)


def _pq_body(e_ref, a_ref, b_ref, p_ref, q_ref):
    e = e_ref[...]
    dn = (((1,), (1,)), ((), ()))
    p_ref[...] = lax.dot_general(e, a_ref[...], dn,
                                 preferred_element_type=jnp.float32)
    q_ref[...] = lax.dot_general(e, b_ref[...], dn,
                                 preferred_element_type=jnp.float32)


def _pq(e, a_tab, b_tab):
    return pl.pallas_call(
        _pq_body,
        grid=(10,),
        in_specs=[
            pl.BlockSpec((1000, E_DIM), lambda i: (i, 0)),
            pl.BlockSpec((KG_R, E_DIM), lambda i: (0, 0)),
            pl.BlockSpec((KG_R, E_DIM), lambda i: (0, 0)),
        ],
        out_specs=[
            pl.BlockSpec((1000, KG_R), lambda i: (i, 0)),
            pl.BlockSpec((1000, KG_R), lambda i: (i, 0)),
        ],
        out_shape=[
            jax.ShapeDtypeStruct((KG_E, KG_R), jnp.float32),
            jax.ShapeDtypeStruct((KG_E, KG_R), jnp.float32),
        ],
    )(e, a_tab, b_tab)


def _rsum_inv_body(rsum_ref, o_ref):
    rs = jnp.sum(rsum_ref[...], axis=0, keepdims=True)
    inv = jnp.where(rs == 0.0, 0.0, 1.0 / rs)
    o_ref[...] = jnp.transpose(inv)


def _rsum_inv(rowsum_partial):
    return pl.pallas_call(
        _rsum_inv_body,
        grid=(1,),
        in_specs=[pl.BlockSpec((NW, KG_E), lambda i: (0, 0))],
        out_specs=pl.BlockSpec((KG_E, 1), lambda i: (0, 0)),
        out_shape=jax.ShapeDtypeStruct((KG_E, 1), jnp.float32),
    )(rowsum_partial)


def _finalize_body(name_ref, outp_ref, rinv_ref, o_ref):
    total = outp_ref[0] + outp_ref[1]
    e_out = jnp.maximum(total * rinv_ref[...], 0.0)
    o_ref[...] = name_ref[...] + BETA1 * e_out


def _finalize(name_embed, out_partial, rowsum_partial):
    rinv = _rsum_inv(rowsum_partial)
    return pl.pallas_call(
        _finalize_body,
        grid=(10,),
        in_specs=[
            pl.BlockSpec((1000, E_DIM), lambda i: (i, 0)),
            pl.BlockSpec((NC, 1000, E_DIM), lambda i: (0, i, 0)),
            pl.BlockSpec((1000, 1), lambda i: (i, 0)),
        ],
        out_specs=pl.BlockSpec((1000, E_DIM), lambda i: (i, 0)),
        out_shape=jax.ShapeDtypeStruct((KG_E, E_DIM), jnp.float32),
    )(name_embed, out_partial, rinv)


_MESH = plsc.VectorSubcoreMesh(core_axis_name="c", subcore_axis_name="s")


_NBUF = 3
_SET_FIELDS = ('src', 'dst', 'rel', 'idxp', 'idxq', 'sp', 'sq', 'rows',
               'sidx', 'semlin', 'semsp', 'semsq', 'semrows', 'semacc')


def _one_set():
    return [
        pltpu.VMEM((C,), jnp.int32),
        pltpu.VMEM((C,), jnp.int32),
        pltpu.VMEM((C,), jnp.int32),
        pltpu.VMEM((C,), jnp.int32),
        pltpu.VMEM((C,), jnp.int32),
        pltpu.VMEM((C,), jnp.float32),
        pltpu.VMEM((C,), jnp.float32),
        pltpu.VMEM((C, E_DIM), jnp.float32),
        pltpu.VMEM((C,), jnp.int32),
        pltpu.SemaphoreType.DMA,
        pltpu.SemaphoreType.DMA,
        pltpu.SemaphoreType.DMA,
        pltpu.SemaphoreType.DMA,
        pltpu.SemaphoreType.DMA,
    ]


@functools.partial(
    pl.kernel,
    out_type=[
        jax.ShapeDtypeStruct((NC, KG_E, E_DIM), jnp.float32),
        jax.ShapeDtypeStruct((NW, KG_E), jnp.float32),
    ],
    mesh=_MESH,
    compiler_params=pltpu.CompilerParams(use_tc_tiling_on_sc=False,
                                         needs_layout_passes=False),
    scratch_types=(
        [pltpu.VMEM((C,), jnp.float32),
         pltpu.VMEM((KG_E,), jnp.float32),
         pltpu.VMEM_SHARED((KG_E, E_DIM), jnp.float32)]
        + _one_set() + _one_set() + _one_set()
    ),
)
def _edge_kernel(pf, qf, ent, src, dst, rel, zeros,
                 out_p, rowsum_p, wv, rsum, acc, *bufs):
    cid = lax.axis_index("c")
    sid = lax.axis_index("s")
    wid = cid * NS + sid

    nf = len(_SET_FIELDS)
    SETS = [dict(zip(_SET_FIELDS, bufs[i * nf:(i + 1) * nf]))
            for i in range(_NBUF)]

    pltpu.sync_copy(zeros.at[pl.ds(sid * RPT, RPT)],
                    acc.at[pl.ds(sid * RPT, RPT)])

    z16 = jnp.zeros((16,), jnp.float32)

    def _zero_rsum(i, carry):
        rsum[pl.ds(i * 16, 16)] = z16
        return carry

    lax.fori_loop(0, KG_E // 16, _zero_rsum, 0)
    plsc.subcore_barrier()

    base0 = wid * EPW

    def _mkidx(buf):
        def body(i, carry):
            s = buf['src'][pl.ds(i * 16, 16)]
            d = buf['dst'][pl.ds(i * 16, 16)]
            r = buf['rel'][pl.ds(i * 16, 16)]
            buf['idxp'][pl.ds(i * 16, 16)] = s * KG_R + r
            buf['idxq'][pl.ds(i * 16, 16)] = d * KG_R + r
            return carry
        lax.fori_loop(0, C // 16, body, 0)

    def _start_lin(c, buf):
        base = base0 + c * C
        pltpu.async_copy(src.at[pl.ds(base, C)], buf['src'], buf['semlin'])
        pltpu.async_copy(dst.at[pl.ds(base, C)], buf['dst'], buf['semlin'])
        pltpu.async_copy(rel.at[pl.ds(base, C)], buf['rel'], buf['semlin'])

    def _wait_lin(buf):
        pltpu.make_async_copy(src.at[pl.ds(0, C)], buf['src'],
                              buf['semlin']).wait()
        pltpu.make_async_copy(dst.at[pl.ds(0, C)], buf['dst'],
                              buf['semlin']).wait()
        pltpu.make_async_copy(rel.at[pl.ds(0, C)], buf['rel'],
                              buf['semlin']).wait()

    def _start_gathers(buf):
        pltpu.async_copy(pf.at[buf['idxp']], buf['sp'], buf['semsp'])
        pltpu.async_copy(qf.at[buf['idxq']], buf['sq'], buf['semsq'])
        pltpu.async_copy(ent.at[buf['dst']], buf['rows'], buf['semrows'])

    def _wait_gathers(buf):
        pltpu.make_async_copy(pf.at[buf['idxp']], buf['sp'],
                              buf['semsp']).wait()
        pltpu.make_async_copy(qf.at[buf['idxq']], buf['sq'],
                              buf['semsq']).wait()
        pltpu.make_async_copy(ent.at[buf['dst']], buf['rows'],
                              buf['semrows']).wait()

    def _do_chunk(c, cur, nxt, nxt2):
        @pl.when(c + 1 < NCHUNK)
        def _():
            _wait_lin(nxt)
            _mkidx(nxt)
            _start_gathers(nxt)

        @pl.when(c + 2 < NCHUNK)
        def _():
            _start_lin(c + 2, nxt2)

        _wait_gathers(cur)

        def _mkw(i, carry2):
            s = cur['sp'][pl.ds(i * 16, 16)] + cur['sq'][pl.ds(i * 16, 16)]
            ls = jnp.where(s > 0.0, s, ALPHA * s)
            wv[pl.ds(i * 16, 16)] = jnp.exp(-ls)
            return carry2

        lax.fori_loop(0, C // 16, _mkw, 0)

        def _scale(i, carry2):
            w16 = wv[pl.ds(i * 16, 16)]
            for l in range(16):
                s = w16[l]
                e = i * 16 + l
                for k in range(E_DIM // 16):
                    cur['rows'][e, pl.ds(k * 16, 16)] = (
                        cur['rows'][e, pl.ds(k * 16, 16)] * s)
            return carry2

        lax.fori_loop(0, C // 16, _scale, 0)

        def _rsacc(i, carry2):
            s16 = cur['src'][pl.ds(i * 16, 16)]
            plsc.addupdate_scatter(rsum, [s16], wv[pl.ds(i * 16, 16)])
            cur['sidx'][pl.ds(i * 16, 16)] = s16
            return carry2

        lax.fori_loop(0, C // 16, _rsacc, 0)

        @pl.when(c >= 1)
        def _():
            pltpu.make_async_copy(nxt2['rows'], acc.at[nxt2['sidx']],
                                  nxt2['semacc']).wait()

        pltpu.async_copy(cur['rows'], acc.at[cur['sidx']], cur['semacc'],
                         add=True)

    S0, S1, S2 = SETS

    _start_lin(0, S0)
    _wait_lin(S0)
    _mkidx(S0)
    _start_gathers(S0)
    _start_lin(1, S1)

    def _triple(j, carry):
        _do_chunk(3 * j, S0, S1, S2)
        _do_chunk(3 * j + 1, S1, S2, S0)
        _do_chunk(3 * j + 2, S2, S0, S1)
        return carry

    lax.fori_loop(0, NCHUNK // 3, _triple, 0)
    for c in range(3 * (NCHUNK // 3), NCHUNK):
        _do_chunk(c, SETS[c % 3], SETS[(c + 1) % 3], SETS[(c + 2) % 3])

    last = SETS[(NCHUNK - 1) % 3]
    pltpu.make_async_copy(last['rows'], acc.at[last['sidx']],
                          last['semacc']).wait()

    plsc.subcore_barrier()

    pltpu.sync_copy(acc.at[pl.ds(sid * RPT, RPT)],
                    out_p.at[cid, pl.ds(sid * RPT, RPT)])
    pltpu.sync_copy(rsum, rowsum_p.at[wid])


def _att_layer(ent, pflat, qflat, src, dst, rel, zeros, name_embed):
    out_p, rowsum_p = _edge_kernel(pflat, qflat, ent, src, dst, rel, zeros)
    return _finalize(name_embed, out_p, rowsum_p)


def kernel(kg_name_embed, eer_adj_index, eer_adj_data, r_head, r_tail,
           kg_name_w, kg_name_b, w_R_Left, w_R_Right, w_atten_r):
    b2d = kg_name_b.reshape(1, E_DIM)
    wal = w_atten_r[:E_DIM, 0].reshape(1, E_DIM)
    war = w_atten_r[E_DIM:, 0].reshape(1, E_DIM)
    src = eer_adj_index[0]
    dst = eer_adj_index[1]
    rel = eer_adj_data
    zeros = jnp.zeros((KG_E, E_DIM), jnp.float32)

    name_embed = _name_proj(kg_name_embed, kg_name_w, b2d)

    def layer(ent):
        le, re = _lr_proj(ent, w_R_Left, w_R_Right)
        a_tab, b_tab = _rel_tables(r_head, r_tail, le, re, wal, war)
        p, q = _pq(ent, a_tab, b_tab)
        return _att_layer(ent, p.reshape(-1), q.reshape(-1),
                          src, dst, rel, zeros, name_embed)

    gat1 = layer(name_embed)
    gat2 = layer(gat1)
    return gat2


# speedup vs baseline: 13.0321x; 1.2232x over previous
import functools

import jax
import jax.numpy as jnp
from jax import lax
from jax.experimental import pallas as pl
from jax.experimental.pallas import tpu as pltpu
from jax.experimental.pallas import tpu_sc as plsc

KG_E = 10000
KG_R = 1000
E_DIM = 128
N_EDGES = 320000
ALPHA = 0.2
BETA1 = 0.3

NC = 2
NS = 16
NW = NC * NS
EPW = N_EDGES // NW
C = 80
NCHUNK = EPW // C
RPT = KG_E // NS


def _name_proj_body(x_ref, w_ref, b_ref, o_ref):
    o_ref[...] = (
        jnp.dot(x_ref[...], w_ref[...], preferred_element_type=jnp.float32)
        + b_ref[...]
    )


def _name_proj(x, w, b2d):
    return pl.pallas_call(
        _name_proj_body,
        grid=(10,),
        in_specs=[
            pl.BlockSpec((1000, 300), lambda i: (i, 0)),
            pl.BlockSpec((300, E_DIM), lambda i: (0, 0)),
            pl.BlockSpec((1, E_DIM), lambda i: (0, 0)),
        ],
        out_specs=pl.BlockSpec((1000, E_DIM), lambda i: (i, 0)),
        out_shape=jax.ShapeDtypeStruct((KG_E, E_DIM), jnp.float32),
    )(x, w, b2d)


def _lr_proj_body(e_ref, wl_ref, wr_ref, le_ref, re_ref):
    e = e_ref[...]
    le_ref[...] = jnp.dot(e, wl_ref[...], preferred_element_type=jnp.float32)
    re_ref[...] = jnp.dot(e, wr_ref[...], preferred_element_type=jnp.float32)


def _lr_proj(e, wl, wr):
    return pl.pallas_call(
        _lr_proj_body,
        grid=(10,),
        in_specs=[
            pl.BlockSpec((1000, E_DIM), lambda i: (i, 0)),
            pl.BlockSpec((E_DIM, E_DIM), lambda i: (0, 0)),
            pl.BlockSpec((E_DIM, E_DIM), lambda i: (0, 0)),
        ],
        out_specs=[
            pl.BlockSpec((1000, E_DIM), lambda i: (i, 0)),
            pl.BlockSpec((1000, E_DIM), lambda i: (i, 0)),
        ],
        out_shape=[
            jax.ShapeDtypeStruct((KG_E, E_DIM), jnp.float32),
            jax.ShapeDtypeStruct((KG_E, E_DIM), jnp.float32),
        ],
    )(e, wl, wr)


def _rel_tables_body(rh_ref, rt_ref, le_ref, re_ref, wal_ref, war_ref,
                     a_ref, b_ref):
    rh = rh_ref[...]
    hs = jnp.sum(rh, axis=1, keepdims=True)
    hinv = jnp.where(hs == 0.0, 0.0, 1.0 / hs)
    lr = jnp.dot(rh, le_ref[...], preferred_element_type=jnp.float32) * hinv
    a_ref[...] = jnp.maximum(lr, 0.0) * wal_ref[...]
    rt = rt_ref[...]
    ts = jnp.sum(rt, axis=1, keepdims=True)
    tinv = jnp.where(ts == 0.0, 0.0, 1.0 / ts)
    rr = jnp.dot(rt, re_ref[...], preferred_element_type=jnp.float32) * tinv
    b_ref[...] = jnp.maximum(rr, 0.0) * war_ref[...]


def _rel_tables(r_head, r_tail, le, re, wal, war):
    return pl.pallas_call(
        _rel_tables_body,
        grid=(5,),
        in_specs=[
            pl.BlockSpec((200, KG_E), lambda i: (i, 0)),
            pl.BlockSpec((200, KG_E), lambda i: (i, 0)),
            pl.BlockSpec((KG_E, E_DIM), lambda i: (0, 0)),
            pl.BlockSpec((KG_E, E_DIM), lambda i: (0, 0)),
            pl.BlockSpec((1, E_DIM), lambda i: (0, 0)),
            pl.BlockSpec((1, E_DIM), lambda i: (0, 0)),
        ],
        out_specs=[
            pl.BlockSpec((200, E_DIM), lambda i: (i, 0)),
            pl.BlockSpec((200, E_DIM), lambda i: (i, 0)),
        ],
        out_shape=[
            jax.ShapeDtypeStruct((KG_R, E_DIM), jnp.float32),
            jax.ShapeDtypeStruct((KG_R, E_DIM), jnp.float32),
        ],
    )(r_head, r_tail, le, re, wal, war)


KG_RP = 1024


def _pq_body(e_ref, a_ref, b_ref, p_ref, q_ref):
    e = e_ref[...]
    dn = (((1,), (1,)), ((), ()))
    for j in range(KG_RP // 128):
        a_j = a_ref[pl.ds(j * 128, 128), :]
        b_j = b_ref[pl.ds(j * 128, 128), :]
        p_ref[:, j, :] = lax.dot_general(e, a_j, dn,
                                         preferred_element_type=jnp.float32)
        q_ref[:, j, :] = lax.dot_general(e, b_j, dn,
                                         preferred_element_type=jnp.float32)


def _pq(e, a_pad, b_pad):
    return pl.pallas_call(
        _pq_body,
        grid=(10,),
        in_specs=[
            pl.BlockSpec((1000, E_DIM), lambda i: (i, 0)),
            pl.BlockSpec((KG_RP, E_DIM), lambda i: (0, 0)),
            pl.BlockSpec((KG_RP, E_DIM), lambda i: (0, 0)),
        ],
        out_specs=[
            pl.BlockSpec((1000, KG_RP // 128, 128), lambda i: (i, 0, 0)),
            pl.BlockSpec((1000, KG_RP // 128, 128), lambda i: (i, 0, 0)),
        ],
        out_shape=[
            jax.ShapeDtypeStruct((KG_E, KG_RP // 128, 128), jnp.float32),
            jax.ShapeDtypeStruct((KG_E, KG_RP // 128, 128), jnp.float32),
        ],
    )(e, a_pad, b_pad)


def _rsum_inv_body(rsum_ref, o_ref):
    rs = jnp.sum(rsum_ref[...], axis=0, keepdims=True)
    inv = jnp.where(rs == 0.0, 0.0, 1.0 / rs)
    o_ref[...] = jnp.transpose(inv)


def _rsum_inv(rowsum_partial):
    return pl.pallas_call(
        _rsum_inv_body,
        grid=(1,),
        in_specs=[pl.BlockSpec((NW, KG_E), lambda i: (0, 0))],
        out_specs=pl.BlockSpec((KG_E, 1), lambda i: (0, 0)),
        out_shape=jax.ShapeDtypeStruct((KG_E, 1), jnp.float32),
    )(rowsum_partial)


def _finalize_body(name_ref, outp_ref, rinv_ref, o_ref):
    total = outp_ref[0] + outp_ref[1]
    e_out = jnp.maximum(total * rinv_ref[...], 0.0)
    o_ref[...] = name_ref[...] + BETA1 * e_out


def _finalize(name_embed, out_partial, rowsum_partial):
    rinv = _rsum_inv(rowsum_partial)
    return pl.pallas_call(
        _finalize_body,
        grid=(10,),
        in_specs=[
            pl.BlockSpec((1000, E_DIM), lambda i: (i, 0)),
            pl.BlockSpec((NC, 1000, E_DIM), lambda i: (0, i, 0)),
            pl.BlockSpec((1000, 1), lambda i: (i, 0)),
        ],
        out_specs=pl.BlockSpec((1000, E_DIM), lambda i: (i, 0)),
        out_shape=jax.ShapeDtypeStruct((KG_E, E_DIM), jnp.float32),
    )(name_embed, out_partial, rinv)


_MESH = plsc.VectorSubcoreMesh(core_axis_name="c", subcore_axis_name="s")


_NBUF = 3
_SET_FIELDS = ('src', 'dst', 'rel', 'idxp', 'idxq', 'sp', 'sq', 'rows',
               'sidx', 'semlin', 'semsp', 'semsq', 'semrows', 'semacc')


def _one_set():
    return [
        pltpu.VMEM((C,), jnp.int32),
        pltpu.VMEM((C,), jnp.int32),
        pltpu.VMEM((C,), jnp.int32),
        pltpu.VMEM((C,), jnp.int32),
        pltpu.VMEM((C,), jnp.int32),
        pltpu.VMEM((C,), jnp.float32),
        pltpu.VMEM((C,), jnp.float32),
        pltpu.VMEM((C, E_DIM), jnp.float32),
        pltpu.VMEM((C,), jnp.int32),
        pltpu.SemaphoreType.DMA,
        pltpu.SemaphoreType.DMA,
        pltpu.SemaphoreType.DMA,
        pltpu.SemaphoreType.DMA,
        pltpu.SemaphoreType.DMA,
    ]


@functools.partial(
    pl.kernel,
    out_type=[
        jax.ShapeDtypeStruct((NC, KG_E, E_DIM), jnp.float32),
        jax.ShapeDtypeStruct((NW, KG_E), jnp.float32),
    ],
    mesh=_MESH,
    compiler_params=pltpu.CompilerParams(use_tc_tiling_on_sc=False,
                                         needs_layout_passes=False),
    scratch_types=(
        [pltpu.VMEM((C,), jnp.float32),
         pltpu.VMEM((KG_E,), jnp.float32),
         pltpu.VMEM_SHARED((KG_E, E_DIM), jnp.float32)]
        + _one_set() + _one_set() + _one_set()
    ),
)
def _edge_kernel(pf, qf, ent, src, dst, rel, zeros,
                 out_p, rowsum_p, wv, rsum, acc, *bufs):
    cid = lax.axis_index("c")
    sid = lax.axis_index("s")
    wid = cid * NS + sid

    nf = len(_SET_FIELDS)
    SETS = [dict(zip(_SET_FIELDS, bufs[i * nf:(i + 1) * nf]))
            for i in range(_NBUF)]

    pltpu.sync_copy(zeros.at[pl.ds(sid * RPT, RPT)],
                    acc.at[pl.ds(sid * RPT, RPT)])

    z16 = jnp.zeros((16,), jnp.float32)

    def _zero_rsum(i, carry):
        rsum[pl.ds(i * 16, 16)] = z16
        return carry

    lax.fori_loop(0, KG_E // 16, _zero_rsum, 0)
    plsc.subcore_barrier()

    base0 = wid * EPW

    def _mkidx(buf):
        def body(i, carry):
            s = buf['src'][pl.ds(i * 16, 16)]
            d = buf['dst'][pl.ds(i * 16, 16)]
            r = buf['rel'][pl.ds(i * 16, 16)]
            buf['idxp'][pl.ds(i * 16, 16)] = s * KG_RP + r
            buf['idxq'][pl.ds(i * 16, 16)] = d * KG_RP + r
            return carry
        lax.fori_loop(0, C // 16, body, 0)

    def _start_lin(c, buf):
        base = base0 + c * C
        pltpu.async_copy(src.at[pl.ds(base, C)], buf['src'], buf['semlin'])
        pltpu.async_copy(dst.at[pl.ds(base, C)], buf['dst'], buf['semlin'])
        pltpu.async_copy(rel.at[pl.ds(base, C)], buf['rel'], buf['semlin'])

    def _wait_lin(buf):
        pltpu.make_async_copy(src.at[pl.ds(0, C)], buf['src'],
                              buf['semlin']).wait()
        pltpu.make_async_copy(dst.at[pl.ds(0, C)], buf['dst'],
                              buf['semlin']).wait()
        pltpu.make_async_copy(rel.at[pl.ds(0, C)], buf['rel'],
                              buf['semlin']).wait()

    def _start_gathers(buf):
        pltpu.async_copy(pf.at[buf['idxp']], buf['sp'], buf['semsp'])
        pltpu.async_copy(qf.at[buf['idxq']], buf['sq'], buf['semsq'])
        pltpu.async_copy(ent.at[buf['dst']], buf['rows'], buf['semrows'])

    def _wait_gathers(buf):
        pltpu.make_async_copy(pf.at[buf['idxp']], buf['sp'],
                              buf['semsp']).wait()
        pltpu.make_async_copy(qf.at[buf['idxq']], buf['sq'],
                              buf['semsq']).wait()
        pltpu.make_async_copy(ent.at[buf['dst']], buf['rows'],
                              buf['semrows']).wait()

    def _do_chunk(c, cur, nxt, nxt2):
        @pl.when(c + 1 < NCHUNK)
        def _():
            _wait_lin(nxt)
            _mkidx(nxt)
            _start_gathers(nxt)

        @pl.when(c + 2 < NCHUNK)
        def _():
            _start_lin(c + 2, nxt2)

        _wait_gathers(cur)

        def _mkw(i, carry2):
            s = cur['sp'][pl.ds(i * 16, 16)] + cur['sq'][pl.ds(i * 16, 16)]
            ls = jnp.where(s > 0.0, s, ALPHA * s)
            wv[pl.ds(i * 16, 16)] = jnp.exp(-ls)
            return carry2

        lax.fori_loop(0, C // 16, _mkw, 0)

        def _scale(i, carry2):
            w16 = wv[pl.ds(i * 16, 16)]
            for l in range(16):
                s = w16[l]
                e = i * 16 + l
                for k in range(E_DIM // 16):
                    cur['rows'][e, pl.ds(k * 16, 16)] = (
                        cur['rows'][e, pl.ds(k * 16, 16)] * s)
            return carry2

        lax.fori_loop(0, C // 16, _scale, 0)

        def _rsacc(i, carry2):
            s16 = cur['src'][pl.ds(i * 16, 16)]
            plsc.addupdate_scatter(rsum, [s16], wv[pl.ds(i * 16, 16)])
            cur['sidx'][pl.ds(i * 16, 16)] = s16
            return carry2

        lax.fori_loop(0, C // 16, _rsacc, 0)

        @pl.when(c >= 1)
        def _():
            pltpu.make_async_copy(nxt2['rows'], acc.at[nxt2['sidx']],
                                  nxt2['semacc']).wait()

        pltpu.async_copy(cur['rows'], acc.at[cur['sidx']], cur['semacc'],
                         add=True)

    S0, S1, S2 = SETS

    _start_lin(0, S0)
    _wait_lin(S0)
    _mkidx(S0)
    _start_gathers(S0)
    _start_lin(1, S1)

    def _triple(j, carry):
        _do_chunk(3 * j, S0, S1, S2)
        _do_chunk(3 * j + 1, S1, S2, S0)
        _do_chunk(3 * j + 2, S2, S0, S1)
        return carry

    lax.fori_loop(0, NCHUNK // 3, _triple, 0)
    for c in range(3 * (NCHUNK // 3), NCHUNK):
        _do_chunk(c, SETS[c % 3], SETS[(c + 1) % 3], SETS[(c + 2) % 3])

    last = SETS[(NCHUNK - 1) % 3]
    pltpu.make_async_copy(last['rows'], acc.at[last['sidx']],
                          last['semacc']).wait()

    plsc.subcore_barrier()

    pltpu.sync_copy(acc.at[pl.ds(sid * RPT, RPT)],
                    out_p.at[cid, pl.ds(sid * RPT, RPT)])
    pltpu.sync_copy(rsum, rowsum_p.at[wid])


def _att_layer(ent, pflat, qflat, src, dst, rel, zeros, name_embed):
    out_p, rowsum_p = _edge_kernel(pflat, qflat, ent, src, dst, rel, zeros)
    return _finalize(name_embed, out_p, rowsum_p)


def kernel(kg_name_embed, eer_adj_index, eer_adj_data, r_head, r_tail,
           kg_name_w, kg_name_b, w_R_Left, w_R_Right, w_atten_r):
    b2d = kg_name_b.reshape(1, E_DIM)
    wal = w_atten_r[:E_DIM, 0].reshape(1, E_DIM)
    war = w_atten_r[E_DIM:, 0].reshape(1, E_DIM)
    src = eer_adj_index[0]
    dst = eer_adj_index[1]
    rel = eer_adj_data
    zeros = jnp.zeros((KG_E, E_DIM), jnp.float32)

    name_embed = _name_proj(kg_name_embed, kg_name_w, b2d)

    relpad = jnp.zeros((KG_RP - KG_R, E_DIM), jnp.float32)

    def layer(ent):
        le, re = _lr_proj(ent, w_R_Left, w_R_Right)
        a_tab, b_tab = _rel_tables(r_head, r_tail, le, re, wal, war)
        p, q = _pq(ent, jnp.concatenate([a_tab, relpad], axis=0),
                   jnp.concatenate([b_tab, relpad], axis=0))
        return _att_layer(ent, p.reshape(-1), q.reshape(-1),
                          src, dst, rel, zeros, name_embed)

    gat1 = layer(name_embed)
    gat2 = layer(gat1)
    return gat2
